# baseline (speedup 1.0000x reference)
import functools

import jax
import jax.numpy as jnp
from jax import lax
from jax.experimental import pallas as pl
from jax.experimental.pallas import tpu as pltpu
from jax.experimental.pallas import tpu_sc as plsc

NB = 128
C = 384
H = W = 28
NROW = H * W * NB
NC, NS, L = 2, 16, 16
NW = NC * NS
RPW = NROW // NW
R = 32
NCHUNK = RPW // R
NJ = C // L


_mesh = plsc.VectorSubcoreMesh(
    core_axis_name="c", subcore_axis_name="s", num_cores=NC, num_subcores=NS
)


@functools.partial(
    pl.kernel,
    mesh=_mesh,
    out_type=jax.ShapeDtypeStruct((NROW, C), jnp.float32),
    scratch_types=[
        pltpu.VMEM((C,), jnp.int32),
        pltpu.VMEM((4 * R, C), jnp.float32),
        pltpu.VMEM((4 * R, C), jnp.float32),
        pltpu.SemaphoreType.DMA,
        pltpu.SemaphoreType.DMA,
    ],
    compiler_params=pltpu.CompilerParams(
        needs_layout_passes=False, disable_bounds_checks=True
    ),
)
def _shuffle(in_hbm, idx_hbm, out_hbm, perm_v, ibuf, obuf, isem, osem):
    wid = lax.axis_index("s") * NC + lax.axis_index("c")
    row0 = wid * RPW

    pltpu.sync_copy(idx_hbm, perm_v)
    pj = [perm_v[pl.ds(j * L, L)] for j in range(NJ)]
    lanes = lax.iota(jnp.int32, L)

    def stream_in(g):
        off = (g % 4) * R
        return pltpu.make_async_copy(
            in_hbm.at[pl.ds(row0 + g * R, R)], ibuf.at[pl.ds(off, R)], isem
        )

    def stream_out(g):
        off = (g % 4) * R
        return pltpu.make_async_copy(
            obuf.at[pl.ds(off, R)], out_hbm.at[pl.ds(row0 + g * R, R)], osem
        )

    stream_in(0).start()
    stream_in(1).start()
    stream_in(2).start()
    stream_in(3).start()

    def chunk_body(g, carry):
        off = (g % 4) * R
        stream_in(g).wait()

        @pl.when(g >= 4)
        def _drain():
            stream_out(g - 4).wait()

        for j0 in (0, NJ // 2):

            @plsc.parallel_loop(0, R, unroll=4)
            def row_body(r):
                rv = jnp.full((L,), off + r, jnp.int32)
                for j in range(j0, j0 + NJ // 2):
                    v = plsc.load_gather(ibuf, [rv, pj[j]])
                    obuf[off + r, pl.ds(j * L, L)] = v

        stream_out(g).start()

        @pl.when(g + 4 < NCHUNK)
        def _prefetch():
            stream_in(g + 4).start()

        return carry

    lax.fori_loop(0, NCHUNK, chunk_body, 0)
    stream_out(NCHUNK - 4).wait()
    stream_out(NCHUNK - 3).wait()
    stream_out(NCHUNK - 2).wait()
    stream_out(NCHUNK - 1).wait()


def kernel(input, indices):
    x = jnp.transpose(input, (2, 3, 0, 1)).reshape(NROW, C)
    out = _shuffle(x, indices.astype(jnp.int32))
    return jnp.transpose(out.reshape(H, W, NB, C), (2, 3, 0, 1))


# speedup vs baseline: 1.0047x; 1.0047x over previous
import functools

import jax
import jax.numpy as jnp
from jax import lax
from jax.experimental import pallas as pl
from jax.experimental.pallas import tpu as pltpu
from jax.experimental.pallas import tpu_sc as plsc

NB = 128
C = 384
H = W = 28
NROW = H * W * NB
NC, NS, L = 2, 16, 16
NW = NC * NS
RPW = NROW // NW
R = 56
NCHUNK = RPW // R
NJ = C // L


_mesh = plsc.VectorSubcoreMesh(
    core_axis_name="c", subcore_axis_name="s", num_cores=NC, num_subcores=NS
)


@functools.partial(
    pl.kernel,
    mesh=_mesh,
    out_type=jax.ShapeDtypeStruct((NROW, C), jnp.float32),
    scratch_types=[
        pltpu.VMEM((C,), jnp.int32),
        pltpu.VMEM((3 * R, C), jnp.float32),
        pltpu.VMEM((3 * R, C), jnp.float32),
        pltpu.SemaphoreType.DMA,
        pltpu.SemaphoreType.DMA,
    ],
    compiler_params=pltpu.CompilerParams(
        needs_layout_passes=False, disable_bounds_checks=True
    ),
)
def _shuffle(in_hbm, idx_hbm, out_hbm, perm_v, ibuf, obuf, isem, osem):
    wid = lax.axis_index("s") * NC + lax.axis_index("c")
    row0 = wid * RPW

    pltpu.sync_copy(idx_hbm, perm_v)
    pj = [perm_v[pl.ds(j * L, L)] for j in range(NJ)]

    def stream_in(g):
        off = (g % 3) * R
        return pltpu.make_async_copy(
            in_hbm.at[pl.ds(row0 + g * R, R)], ibuf.at[pl.ds(off, R)], isem
        )

    def stream_out(g):
        off = (g % 3) * R
        return pltpu.make_async_copy(
            obuf.at[pl.ds(off, R)], out_hbm.at[pl.ds(row0 + g * R, R)], osem
        )

    stream_in(0).start()
    stream_in(1).start()
    stream_in(2).start()

    def chunk_body(g, carry):
        off = (g % 3) * R
        stream_in(g).wait()

        @pl.when(g >= 3)
        def _drain():
            stream_out(g - 3).wait()

        for j0 in (0, NJ // 2):

            @plsc.parallel_loop(0, R, unroll=4)
            def row_body(r):
                rv = jnp.full((L,), off + r, jnp.int32)
                for j in range(j0, j0 + NJ // 2):
                    v = plsc.load_gather(ibuf, [rv, pj[j]])
                    obuf[off + r, pl.ds(j * L, L)] = v

        stream_out(g).start()

        @pl.when(g + 3 < NCHUNK)
        def _prefetch():
            stream_in(g + 3).start()

        return carry

    lax.fori_loop(0, NCHUNK, chunk_body, 0)
    stream_out(NCHUNK - 3).wait()
    stream_out(NCHUNK - 2).wait()
    stream_out(NCHUNK - 1).wait()


def kernel(input, indices):
    x = jnp.transpose(input, (2, 3, 0, 1)).reshape(NROW, C)
    out = _shuffle(x, indices.astype(jnp.int32))
    return jnp.transpose(out.reshape(H, W, NB, C), (2, 3, 0, 1))
